# Initial kernel scaffold; baseline (speedup 1.0000x reference)
import jax
import jax.numpy as jnp
from jax.experimental import pallas as pl


def kernel(x, edge_index, W1, b1, W2, b2, W3, b3):
    raise NotImplementedError("write your pallas kernel here")


# speedup vs baseline: 10.9491x; 10.9491x over previous
import functools

import jax
import jax.numpy as jnp
from jax import lax
from jax.experimental import pallas as pl
from jax.experimental.pallas import tpu as pltpu
from jax.experimental.pallas import tpu_sc as plsc

N = 10000
H = 64
FW = 128
NC = 2
NS = 16
NW = NC * NS
CHUNK = 256
SUB = 128
NSUB = CHUNK // SUB
N_PAD = 10112
STRIP = N_PAD // NS

_MESH = plsc.VectorSubcoreMesh(core_axis_name="c", subcore_axis_name="s")


def _scatter_body(nchunks, u_hbm, src_hbm, dst_hbm, zeros_hbm,
                  p0_hbm, p1_hbm, idx_s, idx_d, rows, acc, sem):
    c = lax.axis_index("c")
    s = lax.axis_index("s")
    wid = c * NS + s
    pltpu.sync_copy(zeros_hbm.at[pl.ds(s * STRIP, STRIP)],
                    acc.at[pl.ds(s * STRIP, STRIP)])
    plsc.subcore_barrier()

    rows_per_chunk = CHUNK // SUB

    def chunk(i, carry):
        rb = (wid * nchunks + i) * rows_per_chunk
        pltpu.sync_copy(src_hbm.at[pl.ds(rb, rows_per_chunk)], idx_s)
        pltpu.sync_copy(dst_hbm.at[pl.ds(rb, rows_per_chunk)], idx_d)
        cps = [pltpu.async_copy(u_hbm.at[idx_s.at[j]],
                                rows.at[pl.ds(j * SUB, SUB)], sem)
               for j in range(NSUB)]
        for cp in cps:
            cp.wait()
        for j in range(NSUB):
            pltpu.sync_copy(rows.at[pl.ds(j * SUB, SUB)],
                            acc.at[idx_d.at[j]], add=True)
        return carry

    lax.fori_loop(0, nchunks, chunk, 0)
    plsc.subcore_barrier()

    @pl.when(c == 0)
    def _():
        pltpu.sync_copy(acc.at[pl.ds(s * STRIP, STRIP)],
                        p0_hbm.at[pl.ds(s * STRIP, STRIP)])

    @pl.when(c == 1)
    def _():
        pltpu.sync_copy(acc.at[pl.ds(s * STRIP, STRIP)],
                        p1_hbm.at[pl.ds(s * STRIP, STRIP)])


def _degree_body(nchunks, dst_hbm, zeros_hbm, ones_hbm,
                 d0_hbm, d1_hbm, idx_d, ones_v, acc, sem):
    c = lax.axis_index("c")
    s = lax.axis_index("s")
    wid = c * NS + s
    pltpu.sync_copy(zeros_hbm.at[pl.ds(s * STRIP, STRIP)],
                    acc.at[pl.ds(s * STRIP, STRIP)])
    pltpu.sync_copy(ones_hbm, ones_v)
    plsc.subcore_barrier()

    rows_per_chunk = CHUNK // SUB

    def chunk(i, carry):
        rb = (wid * nchunks + i) * rows_per_chunk
        pltpu.sync_copy(dst_hbm.at[pl.ds(rb, rows_per_chunk)], idx_d)
        for j in range(NSUB):
            pltpu.sync_copy(ones_v, acc.at[idx_d.at[j]], add=True)
        return carry

    lax.fori_loop(0, nchunks, chunk, 0)
    plsc.subcore_barrier()

    @pl.when(c == 0)
    def _():
        pltpu.sync_copy(acc.at[pl.ds(s * STRIP, STRIP)],
                        d0_hbm.at[pl.ds(s * STRIP, STRIP)])

    @pl.when(c == 1)
    def _():
        pltpu.sync_copy(acc.at[pl.ds(s * STRIP, STRIP)],
                        d1_hbm.at[pl.ds(s * STRIP, STRIP)])


def _sc_scatter(u, src2d, dst2d, zeros, nchunks):
    f = pl.kernel(
        functools.partial(_scatter_body, nchunks),
        out_type=(jax.ShapeDtypeStruct((N_PAD, FW), jnp.float32),
                  jax.ShapeDtypeStruct((N_PAD, FW), jnp.float32)),
        mesh=_MESH,
        scratch_types=[
            pltpu.VMEM((NSUB, SUB), jnp.int32),
            pltpu.VMEM((NSUB, SUB), jnp.int32),
            pltpu.VMEM((CHUNK, FW), jnp.float32),
            pltpu.VMEM_SHARED((N_PAD, FW), jnp.float32),
            pltpu.SemaphoreType.DMA,
        ],
    )
    return f(u, src2d, dst2d, zeros)


def _sc_degree(dst2d, zeros, ones, nchunks):
    f = pl.kernel(
        functools.partial(_degree_body, nchunks),
        out_type=(jax.ShapeDtypeStruct((N_PAD, FW), jnp.float32),
                  jax.ShapeDtypeStruct((N_PAD, FW), jnp.float32)),
        mesh=_MESH,
        scratch_types=[
            pltpu.VMEM((NSUB, SUB), jnp.int32),
            pltpu.VMEM((SUB, FW), jnp.float32),
            pltpu.VMEM_SHARED((N_PAD, FW), jnp.float32),
            pltpu.SemaphoreType.DMA,
        ],
    )
    return f(dst2d, zeros, ones)


BLK = 1000


def _tc_first_body(x_ref, w_ref, d0_ref, d1_ref, u_ref, dinv_ref):
    deg = 1.0 + d0_ref[...] + d1_ref[...]
    dinv = lax.rsqrt(deg)
    dinv_ref[...] = dinv
    u_ref[...] = dinv * jnp.dot(x_ref[...], w_ref[...],
                                preferred_element_type=jnp.float32)


def _tc_mid_body(p0_ref, p1_ref, u_ref, dinv_ref, w_ref, b_ref, o_ref):
    dinv = dinv_ref[...]
    h = dinv * (p0_ref[...] + p1_ref[...] + u_ref[...]) + b_ref[...]
    h = jnp.maximum(h, 0.0)
    o_ref[...] = dinv * jnp.dot(h, w_ref[...],
                                preferred_element_type=jnp.float32)


def _tc_final_body(p0_ref, p1_ref, u_ref, dinv_ref, b_ref, o_ref):
    dinv = dinv_ref[...]
    z = dinv * (p0_ref[...] + p1_ref[...] + u_ref[...]) + b_ref[...]
    o_ref[...] = z[:, :H]


def _row_spec(w):
    return pl.BlockSpec((BLK, w), lambda i: (i, 0))


def _full_spec(r, w):
    return pl.BlockSpec((r, w), lambda i: (0, 0))


def _tc_first(xp, w1p, d0, d1):
    return pl.pallas_call(
        _tc_first_body,
        grid=(N // BLK,),
        in_specs=[_row_spec(8), _full_spec(8, FW), _row_spec(FW),
                  _row_spec(FW)],
        out_specs=(_row_spec(FW), _row_spec(FW)),
        out_shape=(jax.ShapeDtypeStruct((N, FW), jnp.float32),
                   jax.ShapeDtypeStruct((N, FW), jnp.float32)),
    )(xp, w1p, d0, d1)


def _tc_mid(p0, p1, u, dinv, w, b2d):
    return pl.pallas_call(
        _tc_mid_body,
        grid=(N // BLK,),
        in_specs=[_row_spec(FW), _row_spec(FW), _row_spec(FW), _row_spec(FW),
                  _full_spec(FW, FW), _full_spec(1, FW)],
        out_specs=_row_spec(FW),
        out_shape=jax.ShapeDtypeStruct((N, FW), jnp.float32),
    )(p0, p1, u, dinv, w, b2d)


def _tc_final(p0, p1, u, dinv, b2d):
    return pl.pallas_call(
        _tc_final_body,
        grid=(N // BLK,),
        in_specs=[_row_spec(FW), _row_spec(FW), _row_spec(FW), _row_spec(FW),
                  _full_spec(1, FW)],
        out_specs=_row_spec(H),
        out_shape=jax.ShapeDtypeStruct((N, H), jnp.float32),
    )(p0, p1, u, dinv, b2d)


def _pad_w(w):
    return jnp.pad(w, ((0, FW - w.shape[0]), (0, FW - w.shape[1])))


def _pad_b(b):
    return jnp.pad(b, (0, FW - b.shape[0])).reshape(1, FW)


def kernel(x, edge_index, W1, b1, W2, b2, W3, b3):
    E = edge_index.shape[1]
    nchunks = -(-E // (NW * CHUNK))
    e_pad = nchunks * NW * CHUNK
    src = edge_index[0].astype(jnp.int32)
    dst = edge_index[1].astype(jnp.int32)
    src = jnp.concatenate([src, jnp.zeros((e_pad - E,), jnp.int32)])
    dst = jnp.concatenate([dst, jnp.full((e_pad - E,), N, jnp.int32)])
    src2d = src.reshape(e_pad // SUB, SUB)
    dst2d = dst.reshape(e_pad // SUB, SUB)

    zeros = jnp.zeros((N_PAD, FW), jnp.float32)
    ones = jnp.ones((SUB, FW), jnp.float32)
    xp = jnp.pad(x, ((0, 0), (0, 8 - x.shape[1])))
    w1p = jnp.pad(W1, ((0, 8 - W1.shape[0]), (0, FW - W1.shape[1])))

    d0, d1 = _sc_degree(dst2d, zeros, ones, nchunks)

    u1, dinv = _tc_first(xp, w1p, d0, d1)
    a0, a1 = _sc_scatter(u1, src2d, dst2d, zeros, nchunks)
    u2 = _tc_mid(a0, a1, u1, dinv, _pad_w(W2), _pad_b(b1))
    b0, b1_ = _sc_scatter(u2, src2d, dst2d, zeros, nchunks)
    u3 = _tc_mid(b0, b1_, u2, dinv, _pad_w(W3), _pad_b(b2))
    c0, c1 = _sc_scatter(u3, src2d, dst2d, zeros, nchunks)
    return _tc_final(c0, c1, u3, dinv, _pad_b(b3))


# speedup vs baseline: 26.8572x; 2.4529x over previous
import functools

import jax
import jax.numpy as jnp
from jax import lax
from jax.experimental import pallas as pl
from jax.experimental.pallas import tpu as pltpu
from jax.experimental.pallas import tpu_sc as plsc

N = 10000
H = 64
FW = 128
NC = 2
NS = 16
NW = NC * NS
SUB = 128
GRP = 16
N_PAD = 10112
STRIP = N_PAD // NS

_MESH = plsc.VectorSubcoreMesh(core_axis_name="c", subcore_axis_name="s")


def _scatter_body(ngroups, u_hbm, src_hbm, dst_hbm, zeros_hbm,
                  p0_hbm, p1_hbm, idx_s, idx_d, rows0, rows1, acc,
                  sem0, sem1):
    c = lax.axis_index("c")
    s = lax.axis_index("s")
    wid = c * NS + s
    pltpu.sync_copy(zeros_hbm.at[pl.ds(s * STRIP, STRIP)],
                    acc.at[pl.ds(s * STRIP, STRIP)])
    plsc.subcore_barrier()

    def drain(buf, sem):
        pltpu.make_async_copy(u_hbm.at[idx_s.at[0]], buf, sem).wait()

    def group(g, carry):
        base = (wid * ngroups + g) * GRP
        pltpu.sync_copy(src_hbm.at[pl.ds(base, GRP)], idx_s)
        pltpu.sync_copy(dst_hbm.at[pl.ds(base, GRP)], idx_d)
        pltpu.async_copy(u_hbm.at[idx_s.at[0]], rows0, sem0)

        def pair(p, carry):
            pltpu.async_copy(u_hbm.at[idx_s.at[2 * p + 1]], rows1, sem1)
            drain(rows0, sem0)
            pltpu.sync_copy(rows0, acc.at[idx_d.at[2 * p]], add=True)
            pltpu.async_copy(u_hbm.at[idx_s.at[2 * p + 2]], rows0, sem0)
            drain(rows1, sem1)
            pltpu.sync_copy(rows1, acc.at[idx_d.at[2 * p + 1]], add=True)
            return carry

        lax.fori_loop(0, GRP // 2 - 1, pair, 0)
        pltpu.async_copy(u_hbm.at[idx_s.at[GRP - 1]], rows1, sem1)
        drain(rows0, sem0)
        pltpu.sync_copy(rows0, acc.at[idx_d.at[GRP - 2]], add=True)
        drain(rows1, sem1)
        pltpu.sync_copy(rows1, acc.at[idx_d.at[GRP - 1]], add=True)
        return carry

    lax.fori_loop(0, ngroups, group, 0)
    plsc.subcore_barrier()

    @pl.when(c == 0)
    def _():
        pltpu.sync_copy(acc.at[pl.ds(s * STRIP, STRIP)],
                        p0_hbm.at[pl.ds(s * STRIP, STRIP)])

    @pl.when(c == 1)
    def _():
        pltpu.sync_copy(acc.at[pl.ds(s * STRIP, STRIP)],
                        p1_hbm.at[pl.ds(s * STRIP, STRIP)])


def _degree_body(ngroups, dst_hbm, zeros_hbm, ones_hbm,
                 d0_hbm, d1_hbm, idx_d, ones_v, acc, sem):
    c = lax.axis_index("c")
    s = lax.axis_index("s")
    wid = c * NS + s
    pltpu.sync_copy(zeros_hbm.at[pl.ds(s * STRIP, STRIP)],
                    acc.at[pl.ds(s * STRIP, STRIP)])
    pltpu.sync_copy(ones_hbm, ones_v)
    plsc.subcore_barrier()

    def group(g, carry):
        base = (wid * ngroups + g) * GRP
        pltpu.sync_copy(dst_hbm.at[pl.ds(base, GRP)], idx_d)

        def chunk(k, carry):
            pltpu.sync_copy(ones_v, acc.at[idx_d.at[k]], add=True)
            return carry

        lax.fori_loop(0, GRP, chunk, 0)
        return carry

    lax.fori_loop(0, ngroups, group, 0)
    plsc.subcore_barrier()

    @pl.when(c == 0)
    def _():
        pltpu.sync_copy(acc.at[pl.ds(s * STRIP, STRIP)],
                        d0_hbm.at[pl.ds(s * STRIP, STRIP)])

    @pl.when(c == 1)
    def _():
        pltpu.sync_copy(acc.at[pl.ds(s * STRIP, STRIP)],
                        d1_hbm.at[pl.ds(s * STRIP, STRIP)])


def _sc_scatter(u, src2d, dst2d, zeros, ngroups):
    f = pl.kernel(
        functools.partial(_scatter_body, ngroups),
        out_type=(jax.ShapeDtypeStruct((N_PAD, FW), jnp.float32),
                  jax.ShapeDtypeStruct((N_PAD, FW), jnp.float32)),
        mesh=_MESH,
        scratch_types=[
            pltpu.VMEM((GRP, SUB), jnp.int32),
            pltpu.VMEM((GRP, SUB), jnp.int32),
            pltpu.VMEM((SUB, FW), jnp.float32),
            pltpu.VMEM((SUB, FW), jnp.float32),
            pltpu.VMEM_SHARED((N_PAD, FW), jnp.float32),
            pltpu.SemaphoreType.DMA,
            pltpu.SemaphoreType.DMA,
        ],
    )
    return f(u, src2d, dst2d, zeros)


def _sc_degree(dst2d, zeros, ones, ngroups):
    f = pl.kernel(
        functools.partial(_degree_body, ngroups),
        out_type=(jax.ShapeDtypeStruct((N_PAD, FW), jnp.float32),
                  jax.ShapeDtypeStruct((N_PAD, FW), jnp.float32)),
        mesh=_MESH,
        scratch_types=[
            pltpu.VMEM((GRP, SUB), jnp.int32),
            pltpu.VMEM((SUB, FW), jnp.float32),
            pltpu.VMEM_SHARED((N_PAD, FW), jnp.float32),
            pltpu.SemaphoreType.DMA,
        ],
    )
    return f(dst2d, zeros, ones)


BLK = 1000


def _tc_first_body(x_ref, w_ref, d0_ref, d1_ref, u_ref, dinv_ref):
    deg = 1.0 + d0_ref[...] + d1_ref[...]
    dinv = lax.rsqrt(deg)
    dinv_ref[...] = dinv
    u_ref[...] = dinv * jnp.dot(x_ref[...], w_ref[...],
                                preferred_element_type=jnp.float32)


def _tc_mid_body(p0_ref, p1_ref, u_ref, dinv_ref, w_ref, b_ref, o_ref):
    dinv = dinv_ref[...]
    h = dinv * (p0_ref[...] + p1_ref[...] + u_ref[...]) + b_ref[...]
    h = jnp.maximum(h, 0.0)
    o_ref[...] = dinv * jnp.dot(h, w_ref[...],
                                preferred_element_type=jnp.float32)


def _tc_final_body(p0_ref, p1_ref, u_ref, dinv_ref, b_ref, o_ref):
    dinv = dinv_ref[...]
    z = dinv * (p0_ref[...] + p1_ref[...] + u_ref[...]) + b_ref[...]
    o_ref[...] = z[:, :H]


def _row_spec(w):
    return pl.BlockSpec((BLK, w), lambda i: (i, 0))


def _full_spec(r, w):
    return pl.BlockSpec((r, w), lambda i: (0, 0))


def _tc_first(xp, w1p, d0, d1):
    return pl.pallas_call(
        _tc_first_body,
        grid=(N // BLK,),
        in_specs=[_row_spec(8), _full_spec(8, FW), _row_spec(FW),
                  _row_spec(FW)],
        out_specs=(_row_spec(FW), _row_spec(FW)),
        out_shape=(jax.ShapeDtypeStruct((N, FW), jnp.float32),
                   jax.ShapeDtypeStruct((N, FW), jnp.float32)),
    )(xp, w1p, d0, d1)


def _tc_mid(p0, p1, u, dinv, w, b2d):
    return pl.pallas_call(
        _tc_mid_body,
        grid=(N // BLK,),
        in_specs=[_row_spec(FW), _row_spec(FW), _row_spec(FW), _row_spec(FW),
                  _full_spec(FW, FW), _full_spec(1, FW)],
        out_specs=_row_spec(FW),
        out_shape=jax.ShapeDtypeStruct((N, FW), jnp.float32),
    )(p0, p1, u, dinv, w, b2d)


def _tc_final(p0, p1, u, dinv, b2d):
    return pl.pallas_call(
        _tc_final_body,
        grid=(N // BLK,),
        in_specs=[_row_spec(FW), _row_spec(FW), _row_spec(FW), _row_spec(FW),
                  _full_spec(1, FW)],
        out_specs=_row_spec(H),
        out_shape=jax.ShapeDtypeStruct((N, H), jnp.float32),
    )(p0, p1, u, dinv, b2d)


def _pad_w(w):
    return jnp.pad(w, ((0, FW - w.shape[0]), (0, FW - w.shape[1])))


def _pad_b(b):
    return jnp.pad(b, (0, FW - b.shape[0])).reshape(1, FW)


def kernel(x, edge_index, W1, b1, W2, b2, W3, b3):
    E = edge_index.shape[1]
    ngroups = -(-E // (NW * GRP * SUB))
    e_pad = ngroups * NW * GRP * SUB
    src = edge_index[0].astype(jnp.int32)
    dst = edge_index[1].astype(jnp.int32)
    npad = e_pad - E
    pad_iota = jnp.arange(npad, dtype=jnp.int32)
    src = jnp.concatenate([src, pad_iota % N])
    dst = jnp.concatenate([dst, N + pad_iota % (N_PAD - N)])
    src2d = src.reshape(e_pad // SUB, SUB)
    dst2d = dst.reshape(e_pad // SUB, SUB)

    zeros = jnp.zeros((N_PAD, FW), jnp.float32)
    ones = jnp.ones((SUB, FW), jnp.float32)
    xp = jnp.pad(x, ((0, 0), (0, 8 - x.shape[1])))
    w1p = jnp.pad(W1, ((0, 8 - W1.shape[0]), (0, FW - W1.shape[1])))

    d0, d1 = _sc_degree(dst2d, zeros, ones, ngroups)

    u1, dinv = _tc_first(xp, w1p, d0, d1)
    a0, a1 = _sc_scatter(u1, src2d, dst2d, zeros, ngroups)
    u2 = _tc_mid(a0, a1, u1, dinv, _pad_w(W2), _pad_b(b1))
    b0, b1_ = _sc_scatter(u2, src2d, dst2d, zeros, ngroups)
    u3 = _tc_mid(b0, b1_, u2, dinv, _pad_w(W3), _pad_b(b2))
    c0, c1 = _sc_scatter(u3, src2d, dst2d, zeros, ngroups)
    return _tc_final(c0, c1, u3, dinv, _pad_b(b3))
